# baseline (speedup 1.0000x reference)
import jax
import jax.numpy as jnp
from jax import lax
from jax.experimental import pallas as pl
from jax.experimental.pallas import tpu as pltpu, tpu_sc as plsc

N_ROWS = 20000
NUM_CLASSES = 80
N_KEEP = 1000
CONF = 0.25
BOX_D = 4

NUM_SUBCORES = 16
LANES = 16
ROWS_PER_TILE = 64
PR_CHUNK = ROWS_PER_TILE * NUM_CLASSES
BX_CHUNK = ROWS_PER_TILE * BOX_D


def _sc_body(pr_hbm, bx_hbm, out_hbm,
             pr_v, bx_v, stage_v, li_v, lf_v, out_v, sh_i, sh_f):
    c = lax.axis_index("c")
    s = lax.axis_index("s")
    base_row = s * ROWS_PER_TILE

    pltpu.sync_copy(pr_hbm.at[pl.ds(base_row * NUM_CLASSES, PR_CHUNK)], pr_v)
    pltpu.sync_copy(bx_hbm.at[pl.ds(base_row * BOX_D, BX_CHUNK)], bx_v)

    scores = []
    fail = jnp.int32(N_KEEP)
    for r in range(ROWS_PER_TILE):
        off = r * NUM_CLASSES
        m = pr_v[pl.ds(off, LANES)]
        for k in range(1, NUM_CLASSES // LANES):
            m = jnp.maximum(m, pr_v[pl.ds(off + k * LANES, LANES)])
        sc = jnp.max(m)
        scores.append(sc)
        gid = base_row + r
        failc = jnp.where((sc < CONF) & (gid < N_KEEP), gid, N_KEEP)
        fail = jnp.minimum(fail, failc)

    stage_v[...] = jnp.zeros((LANES,), jnp.int32) + fail
    pltpu.sync_copy(stage_v, sh_i.at[pl.ds(s * LANES, LANES)])
    plsc.subcore_barrier()
    pltpu.sync_copy(sh_i, li_v)
    acc = li_v[pl.ds(0, LANES)]
    for r in range(1, NUM_SUBCORES):
        acc = jnp.minimum(acc, li_v[pl.ds(r * LANES, LANES)])
    gfail = jnp.min(acc)

    ssum = jnp.float32(0.0)
    for r in range(ROWS_PER_TILE):
        gid = base_row + r
        ssum = ssum + jnp.where(gid < gfail, scores[r], jnp.float32(0.0))

    iota = lax.broadcasted_iota(jnp.int32, (LANES,), 0)
    bacc = jnp.zeros((LANES,), jnp.float32)
    for v in range(BX_CHUNK // LANES):
        bv = bx_v[pl.ds(v * LANES, LANES)]
        rows = base_row + v * BOX_D + (iota >> 2)
        bacc = bacc + jnp.where(rows < gfail, bv, jnp.float32(0.0))
    part = ssum + jnp.sum(bacc)

    out_v[...] = jnp.zeros((LANES,), jnp.float32) + part
    pltpu.sync_copy(out_v, sh_f.at[pl.ds(s * LANES, LANES)])
    plsc.subcore_barrier()

    @pl.when((c == 0) & (s == 0))
    def _():
        pltpu.sync_copy(sh_f, lf_v)
        a = lf_v[pl.ds(0, LANES)]
        for r in range(1, NUM_SUBCORES):
            a = a + lf_v[pl.ds(r * LANES, LANES)]
        out_v[...] = a
        pltpu.sync_copy(out_v, out_hbm)


@jax.jit
def kernel(post_result, pre_post_boxes):
    pr_flat = post_result.reshape(-1)
    bx_flat = pre_post_boxes.reshape(-1)
    mesh = plsc.VectorSubcoreMesh(core_axis_name="c", subcore_axis_name="s")
    out = pl.kernel(
        _sc_body,
        out_type=jax.ShapeDtypeStruct((LANES,), jnp.float32),
        mesh=mesh,
        compiler_params=pltpu.CompilerParams(needs_layout_passes=False),
        scratch_types=[
            pltpu.VMEM((PR_CHUNK,), jnp.float32),
            pltpu.VMEM((BX_CHUNK,), jnp.float32),
            pltpu.VMEM((LANES,), jnp.int32),
            pltpu.VMEM((NUM_SUBCORES * LANES,), jnp.int32),
            pltpu.VMEM((NUM_SUBCORES * LANES,), jnp.float32),
            pltpu.VMEM((LANES,), jnp.float32),
            pltpu.VMEM_SHARED((NUM_SUBCORES * LANES,), jnp.int32),
            pltpu.VMEM_SHARED((NUM_SUBCORES * LANES,), jnp.float32),
        ],
    )(pr_flat, bx_flat)
    return out[0]


# speedup vs baseline: 1.8557x; 1.8557x over previous
import jax
import jax.numpy as jnp
from jax import lax
from jax.experimental import pallas as pl
from jax.experimental.pallas import tpu as pltpu, tpu_sc as plsc

N_ROWS = 20000
NUM_CLASSES = 80
N_KEEP = 1000
CONF = 0.25
BOX_D = 4

NUM_SUBCORES = 16
LANES = 16
ROWS_PER_TILE = 64


def _sc_body(pr_hbm, bx_hbm, out_hbm,
             pr_v, bx_v, stage_v, li_v, lf_v, out_v, sh_i, sh_f):
    c = lax.axis_index("c")
    s = lax.axis_index("s")
    base_row = s * ROWS_PER_TILE

    pltpu.sync_copy(pr_hbm.at[pl.ds(base_row, ROWS_PER_TILE)], pr_v)
    pltpu.sync_copy(bx_hbm.at[pl.ds(base_row, ROWS_PER_TILE)], bx_v)

    scores = []
    fail = jnp.int32(N_KEEP)
    for r in range(ROWS_PER_TILE):
        m = pr_v[r, pl.ds(0, LANES)]
        for k in range(1, NUM_CLASSES // LANES):
            m = jnp.maximum(m, pr_v[r, pl.ds(k * LANES, LANES)])
        sc = jnp.max(m)
        scores.append(sc)
        gid = base_row + r
        failc = jnp.where((sc < CONF) & (gid < N_KEEP), gid, N_KEEP)
        fail = jnp.minimum(fail, failc)

    stage_v[...] = jnp.zeros((LANES,), jnp.int32) + fail
    pltpu.sync_copy(stage_v, sh_i.at[pl.ds(s * LANES, LANES)])
    plsc.subcore_barrier()
    pltpu.sync_copy(sh_i, li_v)
    acc = li_v[pl.ds(0, LANES)]
    for r in range(1, NUM_SUBCORES):
        acc = jnp.minimum(acc, li_v[pl.ds(r * LANES, LANES)])
    gfail = jnp.min(acc)

    ssum = jnp.float32(0.0)
    for r in range(ROWS_PER_TILE):
        gid = base_row + r
        ssum = ssum + jnp.where(gid < gfail, scores[r], jnp.float32(0.0))

    iota = lax.broadcasted_iota(jnp.int32, (LANES,), 0)
    row_off = iota >> 2
    col_idx = iota & 3
    bacc = jnp.zeros((LANES,), jnp.float32)
    for v in range(ROWS_PER_TILE // 4):
        bv = plsc.load_gather(bx_v, [v * 4 + row_off, col_idx])
        rows = base_row + v * 4 + row_off
        bacc = bacc + jnp.where(rows < gfail, bv, jnp.float32(0.0))
    part = ssum + jnp.sum(bacc)

    out_v[...] = jnp.zeros((LANES,), jnp.float32) + part
    pltpu.sync_copy(out_v, sh_f.at[pl.ds(s * LANES, LANES)])
    plsc.subcore_barrier()

    @pl.when((c == 0) & (s == 0))
    def _():
        pltpu.sync_copy(sh_f, lf_v)
        a = lf_v[pl.ds(0, LANES)]
        for r in range(1, NUM_SUBCORES):
            a = a + lf_v[pl.ds(r * LANES, LANES)]
        out_v[...] = a
        pltpu.sync_copy(out_v, out_hbm)


@jax.jit
def kernel(post_result, pre_post_boxes):
    mesh = plsc.VectorSubcoreMesh(core_axis_name="c", subcore_axis_name="s")
    out = pl.kernel(
        _sc_body,
        out_type=jax.ShapeDtypeStruct((LANES,), jnp.float32),
        mesh=mesh,
        compiler_params=pltpu.CompilerParams(needs_layout_passes=False),
        scratch_types=[
            pltpu.VMEM((ROWS_PER_TILE, NUM_CLASSES), jnp.float32),
            pltpu.VMEM((ROWS_PER_TILE, BOX_D), jnp.float32),
            pltpu.VMEM((LANES,), jnp.int32),
            pltpu.VMEM((NUM_SUBCORES * LANES,), jnp.int32),
            pltpu.VMEM((NUM_SUBCORES * LANES,), jnp.float32),
            pltpu.VMEM((LANES,), jnp.float32),
            pltpu.VMEM_SHARED((NUM_SUBCORES * LANES,), jnp.int32),
            pltpu.VMEM_SHARED((NUM_SUBCORES * LANES,), jnp.float32),
        ],
    )(post_result, pre_post_boxes)
    return out[0]


# speedup vs baseline: 2.8918x; 1.5584x over previous
import jax
import jax.numpy as jnp
from jax import lax
from jax.experimental import pallas as pl
from jax.experimental.pallas import tpu as pltpu, tpu_sc as plsc

N_ROWS = 20000
NUM_CLASSES = 80
N_KEEP = 1000
CONF = 0.25
BOX_D = 4

NUM_SUBCORES = 16
LANES = 16
ROWS_PER_TILE = 64
GROUPS = ROWS_PER_TILE // LANES


def _sc_body(prt_hbm, bxt_hbm, out_hbm,
             pr_v, bx_v, stage_i, stage_f, li_v, lf_v, sh_i, sh_f):
    c = lax.axis_index("c")
    s = lax.axis_index("s")
    base = s * ROWS_PER_TILE
    blk = (s // 2) * (2 * ROWS_PER_TILE)
    half = (s % 2) * ROWS_PER_TILE

    pltpu.sync_copy(prt_hbm.at[:, pl.ds(blk, 2 * ROWS_PER_TILE)], pr_v)
    pltpu.sync_copy(bxt_hbm.at[:, pl.ds(blk, 2 * ROWS_PER_TILE)], bx_v)

    iota = lax.broadcasted_iota(jnp.int32, (LANES,), 0)

    svecs = []
    idvecs = []
    fail_vec = jnp.full((LANES,), N_KEEP, dtype=jnp.int32)
    for g in range(GROUPS):
        sv = pr_v[0, pl.ds(half + g * LANES, LANES)]
        for k in range(1, NUM_CLASSES):
            sv = jnp.maximum(sv, pr_v[k, pl.ds(half + g * LANES, LANES)])
        ids = base + g * LANES + iota
        svecs.append(sv)
        idvecs.append(ids)
        failc = jnp.where((sv < CONF) & (ids < N_KEEP), ids, N_KEEP)
        fail_vec = jnp.minimum(fail_vec, failc)

    stage_i[...] = fail_vec
    pltpu.sync_copy(stage_i, sh_i.at[pl.ds(s * LANES, LANES)])
    plsc.subcore_barrier()
    pltpu.sync_copy(sh_i, li_v)
    acc = li_v[pl.ds(0, LANES)]
    for r in range(1, NUM_SUBCORES):
        acc = jnp.minimum(acc, li_v[pl.ds(r * LANES, LANES)])
    gfail = jnp.min(acc)

    part = jnp.zeros((LANES,), jnp.float32)
    for g in range(GROUPS):
        bsum = bx_v[0, pl.ds(half + g * LANES, LANES)]
        for k in range(1, BOX_D):
            bsum = bsum + bx_v[k, pl.ds(half + g * LANES, LANES)]
        keep = idvecs[g] < gfail
        part = part + jnp.where(keep, svecs[g] + bsum, jnp.float32(0.0))

    stage_f[...] = part
    pltpu.sync_copy(stage_f, sh_f.at[pl.ds(s * LANES, LANES)])
    plsc.subcore_barrier()

    @pl.when((c == 0) & (s == 0))
    def _():
        pltpu.sync_copy(sh_f, lf_v)
        a = lf_v[pl.ds(0, LANES)]
        for r in range(1, NUM_SUBCORES):
            a = a + lf_v[pl.ds(r * LANES, LANES)]
        stage_f[...] = jnp.zeros((LANES,), jnp.float32) + jnp.sum(a)
        pltpu.sync_copy(stage_f, out_hbm)


@jax.jit
def kernel(post_result, pre_post_boxes):
    mesh = plsc.VectorSubcoreMesh(core_axis_name="c", subcore_axis_name="s")
    out = pl.kernel(
        _sc_body,
        out_type=jax.ShapeDtypeStruct((LANES,), jnp.float32),
        mesh=mesh,
        compiler_params=pltpu.CompilerParams(
            needs_layout_passes=False, use_tc_tiling_on_sc=True
        ),
        scratch_types=[
            pltpu.VMEM((NUM_CLASSES, 2 * ROWS_PER_TILE), jnp.float32),
            pltpu.VMEM((BOX_D, 2 * ROWS_PER_TILE), jnp.float32),
            pltpu.VMEM((LANES,), jnp.int32),
            pltpu.VMEM((LANES,), jnp.float32),
            pltpu.VMEM((NUM_SUBCORES * LANES,), jnp.int32),
            pltpu.VMEM((NUM_SUBCORES * LANES,), jnp.float32),
            pltpu.VMEM_SHARED((NUM_SUBCORES * LANES,), jnp.int32),
            pltpu.VMEM_SHARED((NUM_SUBCORES * LANES,), jnp.float32),
        ],
    )(post_result.T, pre_post_boxes.T)
    return out[0]
